# baseline (speedup 1.0000x reference)
import functools

import jax
import jax.numpy as jnp
from jax import lax
from jax.experimental import pallas as pl
from jax.experimental.pallas import tpu as pltpu
from jax.experimental.pallas import tpu_sc as plsc

_LANES = 16


def kernel(input_ids, segment_ids, token_table, seg_table, pos_table):
    B, L = input_ids.shape
    V, D = token_table.shape

    posseg = pos_table[:L] + seg_table[0][None, :]

    NC, NS = 2, 16
    NW = NC * NS
    bpw = B // NW

    mesh = plsc.VectorSubcoreMesh(core_axis_name="c", subcore_axis_name="s")

    @functools.partial(
        pl.kernel,
        mesh=mesh,
        out_type=jax.ShapeDtypeStruct((B, L, D), jnp.float32),
        scratch_types=[
            pltpu.VMEM((bpw, L), jnp.int32),
            pltpu.VMEM((L, D), jnp.float32),
            pltpu.VMEM((L, D), jnp.float32),
            pltpu.VMEM((L, D), jnp.float32),
            pltpu.SemaphoreType.DMA,
            pltpu.SemaphoreType.DMA,
            pltpu.SemaphoreType.DMA,
            pltpu.SemaphoreType.DMA,
            pltpu.SemaphoreType.DMA,
        ],
        compiler_params=pltpu.CompilerParams(use_tc_tiling_on_sc=False),
    )
    def emb_kernel(ids_hbm, posseg_hbm, tok_hbm, out_hbm,
                   idx_v, rows0, rows1, ps_v,
                   gsem0, gsem1, osem0, osem1, psem):
        wid = lax.axis_index("s") * NC + lax.axis_index("c")
        b0 = wid * bpw
        rows = (rows0, rows1)
        gsem = (gsem0, gsem1)
        osem = (osem0, osem1)

        cp_idx = pltpu.async_copy(ids_hbm.at[pl.ds(b0, bpw)], idx_v, psem)
        pltpu.sync_copy(posseg_hbm, ps_v)
        cp_idx.wait()

        def start_gather(c, s):
            pltpu.async_copy(
                tok_hbm.at[idx_v.at[c, pl.ds(0, 128)]],
                rows[s].at[pl.ds(0, 128)],
                gsem[s],
            )
            pltpu.async_copy(
                tok_hbm.at[idx_v.at[c, pl.ds(128, L - 128)]],
                rows[s].at[pl.ds(128, L - 128)],
                gsem[s],
            )

        def wait_gather(s):
            pltpu.make_async_copy(tok_hbm.at[idx_v.at[0, pl.ds(0, 128)]],
                                  rows[s].at[pl.ds(0, 128)], gsem[s]).wait()
            pltpu.make_async_copy(tok_hbm.at[idx_v.at[0, pl.ds(128, L - 128)]],
                                  rows[s].at[pl.ds(128, L - 128)], gsem[s]).wait()

        def wait_out(c, s):
            pltpu.make_async_copy(rows[s], out_hbm.at[c], osem[s]).wait()

        start_gather(0, 0)

        def body(c, carry):
            for s in range(2):
                @pl.when(lax.rem(c, 2) == s)
                def _():
                    nxt = 1 - s

                    @pl.when(c + 1 < bpw)
                    def _():
                        @pl.when(c + 1 >= 2)
                        def _():
                            wait_out(b0 + c - 1, nxt)
                        start_gather(c + 1, nxt)

                    wait_gather(s)

                    def addrow(r, carry2):
                        for j in range(D // _LANES):
                            sl = pl.ds(j * _LANES, _LANES)
                            rows[s][r, sl] = rows[s][r, sl] + ps_v[r, sl]
                        return carry2

                    lax.fori_loop(0, L, addrow, 0, unroll=2)
                    pltpu.async_copy(rows[s], out_hbm.at[b0 + c], osem[s])
            return carry

        lax.fori_loop(0, bpw, body, 0)
        wait_out(b0 + bpw - 2, bpw % 2)
        wait_out(b0 + bpw - 1, (bpw - 1) % 2)

    return emb_kernel(input_ids, posseg, token_table)
